# Initial kernel scaffold; baseline (speedup 1.0000x reference)
import jax
import jax.numpy as jnp
from jax.experimental import pallas as pl


def kernel(x, edge_index, W1, b1, W2, b2):
    raise NotImplementedError("write your pallas kernel here")


# speedup vs baseline: 9.9339x; 9.9339x over previous
import functools

import jax
import jax.numpy as jnp
from jax import lax
from jax.experimental import pallas as pl
from jax.experimental.pallas import tpu as pltpu
from jax.experimental.pallas import tpu_sc as plsc

N = 10000
E = 320000
D = 128
NC = 2
NS = 16
NW = NC * NS
K = 128
CHUNKS = 79
EPW = K * CHUNKS
E_PAD = EPW * NW
ACC_ROWS = 10240
RPS = ACC_ROWS // NS

_mesh = plsc.VectorSubcoreMesh(core_axis_name="c", subcore_axis_name="s")


@functools.partial(
    pl.kernel,
    out_type=jax.ShapeDtypeStruct((NC * ACC_ROWS, 16), jnp.float32),
    mesh=_mesh,
    scratch_types=[
        pltpu.VMEM((K,), jnp.int32),
        pltpu.VMEM((K, 16), jnp.float32),
        pltpu.VMEM_SHARED((ACC_ROWS, 16), jnp.float32),
    ],
)
def _sc_degree(dst_hbm, out_hbm, dst_v, buf_v, acc_sh):
    cid = lax.axis_index("c")
    sid = lax.axis_index("s")
    wid = sid * NC + cid

    def fill(val):
        def step(i, _):
            buf_v[i] = jnp.full((16,), val, jnp.float32)
            return 0
        lax.fori_loop(0, K, step, 0)

    fill(0.0)

    def zero_slice(k, _):
        pltpu.sync_copy(buf_v, acc_sh.at[pl.ds(sid * RPS + k * K, K)])
        return 0

    lax.fori_loop(0, RPS // K, zero_slice, 0)
    fill(1.0)
    plsc.subcore_barrier()

    base = wid * EPW

    def chunk(j, _):
        pltpu.sync_copy(dst_hbm.at[pl.ds(base + j * K, K)], dst_v)
        pltpu.sync_copy(buf_v, acc_sh.at[dst_v], add=True)
        return 0

    lax.fori_loop(0, CHUNKS, chunk, 0)
    plsc.subcore_barrier()
    pltpu.sync_copy(
        acc_sh.at[pl.ds(sid * RPS, RPS)],
        out_hbm.at[pl.ds(cid * ACC_ROWS + sid * RPS, RPS)],
    )


@functools.partial(
    pl.kernel,
    out_type=jax.ShapeDtypeStruct((NC * ACC_ROWS, D), jnp.float32),
    mesh=_mesh,
    scratch_types=[
        pltpu.VMEM((K,), jnp.int32),
        pltpu.VMEM((K,), jnp.int32),
        pltpu.VMEM((K, D), jnp.float32),
        pltpu.VMEM_SHARED((ACC_ROWS, D), jnp.float32),
        pltpu.SemaphoreType.DMA,
    ],
)
def _sc_scatter(y_hbm, src_hbm, dst_hbm, out_hbm, src_v, dst_v, rows_v, acc_sh, sem):
    cid = lax.axis_index("c")
    sid = lax.axis_index("s")
    wid = sid * NC + cid

    def zero_rows(t, _):
        rows_v[t // 8, pl.ds((t % 8) * 16, 16)] = jnp.zeros((16,), jnp.float32)
        return 0

    lax.fori_loop(0, K * (D // 16), zero_rows, 0)

    def zero_slice(k, _):
        pltpu.sync_copy(rows_v, acc_sh.at[pl.ds(sid * RPS + k * K, K)])
        return 0

    lax.fori_loop(0, RPS // K, zero_slice, 0)
    plsc.subcore_barrier()

    base = wid * EPW

    def chunk(j, _):
        pltpu.sync_copy(src_hbm.at[pl.ds(base + j * K, K)], src_v)
        pltpu.sync_copy(dst_hbm.at[pl.ds(base + j * K, K)], dst_v)
        pltpu.async_copy(y_hbm.at[src_v], rows_v, sem).wait()
        pltpu.sync_copy(rows_v, acc_sh.at[dst_v], add=True)
        return 0

    lax.fori_loop(0, CHUNKS, chunk, 0)
    plsc.subcore_barrier()
    pltpu.sync_copy(
        acc_sh.at[pl.ds(sid * RPS, RPS)],
        out_hbm.at[pl.ds(cid * ACC_ROWS + sid * RPS, RPS)],
    )


_ROWS_BLK = 1000
_GRID = N // _ROWS_BLK


def _tc1(x, W1, d0, d1):
    def body(x_ref, w_ref, d0_ref, d1_ref, y_ref):
        dinv = lax.rsqrt(d0_ref[...] + d1_ref[...] + 1.0)
        y_ref[...] = (
            jnp.dot(x_ref[...], w_ref[...], preferred_element_type=jnp.float32)
            * dinv
        )

    return pl.pallas_call(
        body,
        grid=(_GRID,),
        in_specs=[
            pl.BlockSpec((_ROWS_BLK, D), lambda i: (i, 0)),
            pl.BlockSpec((D, D), lambda i: (0, 0)),
            pl.BlockSpec((_ROWS_BLK, 1), lambda i: (i, 0)),
            pl.BlockSpec((_ROWS_BLK, 1), lambda i: (i, 0)),
        ],
        out_specs=pl.BlockSpec((_ROWS_BLK, D), lambda i: (i, 0)),
        out_shape=jax.ShapeDtypeStruct((N, D), jnp.float32),
    )(x, W1, d0, d1)


def _tc2(p0, p1, y1, d0, d1, b1, W2):
    def body(p0_ref, p1_ref, y1_ref, d0_ref, d1_ref, b_ref, w_ref, out_ref):
        dinv = lax.rsqrt(d0_ref[...] + d1_ref[...] + 1.0)
        h = dinv * (p0_ref[...] + p1_ref[...] + y1_ref[...]) + b_ref[...]
        h = jnp.where(h > 0, h, 0.01 * h)
        out_ref[...] = (
            jnp.dot(h, w_ref[...], preferred_element_type=jnp.float32) * dinv
        )

    return pl.pallas_call(
        body,
        grid=(_GRID,),
        in_specs=[
            pl.BlockSpec((_ROWS_BLK, D), lambda i: (i, 0)),
            pl.BlockSpec((_ROWS_BLK, D), lambda i: (i, 0)),
            pl.BlockSpec((_ROWS_BLK, D), lambda i: (i, 0)),
            pl.BlockSpec((_ROWS_BLK, 1), lambda i: (i, 0)),
            pl.BlockSpec((_ROWS_BLK, 1), lambda i: (i, 0)),
            pl.BlockSpec((1, D), lambda i: (0, 0)),
            pl.BlockSpec((D, D), lambda i: (0, 0)),
        ],
        out_specs=pl.BlockSpec((_ROWS_BLK, D), lambda i: (i, 0)),
        out_shape=jax.ShapeDtypeStruct((N, D), jnp.float32),
    )(p0, p1, y1, d0, d1, b1, W2)


def _tc3(q0, q1, y2, d0, d1, b2):
    def body(q0_ref, q1_ref, y2_ref, d0_ref, d1_ref, b_ref, out_ref):
        dinv = lax.rsqrt(d0_ref[...] + d1_ref[...] + 1.0)
        out_ref[...] = dinv * (q0_ref[...] + q1_ref[...] + y2_ref[...]) + b_ref[...]

    return pl.pallas_call(
        body,
        grid=(_GRID,),
        in_specs=[
            pl.BlockSpec((_ROWS_BLK, D), lambda i: (i, 0)),
            pl.BlockSpec((_ROWS_BLK, D), lambda i: (i, 0)),
            pl.BlockSpec((_ROWS_BLK, D), lambda i: (i, 0)),
            pl.BlockSpec((_ROWS_BLK, 1), lambda i: (i, 0)),
            pl.BlockSpec((_ROWS_BLK, 1), lambda i: (i, 0)),
            pl.BlockSpec((1, D), lambda i: (0, 0)),
        ],
        out_specs=pl.BlockSpec((_ROWS_BLK, D), lambda i: (i, 0)),
        out_shape=jax.ShapeDtypeStruct((N, D), jnp.float32),
    )(q0, q1, y2, d0, d1, b2)


def kernel(x, edge_index, W1, b1, W2, b2):
    src = edge_index[0]
    dst = edge_index[1]
    pad = E_PAD - E
    srcp = jnp.concatenate([src, jnp.zeros((pad,), jnp.int32)])
    dstp = jnp.concatenate([dst, jnp.full((pad,), N, jnp.int32)])

    degp = _sc_degree(dstp)
    d0 = degp[:N, :1]
    d1 = degp[ACC_ROWS:ACC_ROWS + N, :1]

    y1 = _tc1(x, W1, d0, d1)

    p = _sc_scatter(y1, srcp, dstp)
    y2 = _tc2(p[:N], p[ACC_ROWS:ACC_ROWS + N], y1, d0, d1,
              b1.reshape(1, D), W2)

    q = _sc_scatter(y2, srcp, dstp)
    return _tc3(q[:N], q[ACC_ROWS:ACC_ROWS + N], y2, d0, d1,
                b2.reshape(1, D))


# speedup vs baseline: 12.5450x; 1.2628x over previous
import functools

import jax
import jax.numpy as jnp
from jax import lax
from jax.experimental import pallas as pl
from jax.experimental.pallas import tpu as pltpu
from jax.experimental.pallas import tpu_sc as plsc

N = 10000
E = 320000
D = 128
NC = 2
NS = 16
NW = NC * NS
K = 128
CHUNKS = 79
EPW = K * CHUNKS
E_PAD = EPW * NW
ACC_ROWS = 10240
RPS = ACC_ROWS // NS

_mesh = plsc.VectorSubcoreMesh(core_axis_name="c", subcore_axis_name="s")


@functools.partial(
    pl.kernel,
    out_type=jax.ShapeDtypeStruct((NC * ACC_ROWS, 16), jnp.float32),
    mesh=_mesh,
    scratch_types=[
        pltpu.VMEM((K,), jnp.int32),
        pltpu.VMEM((K,), jnp.int32),
        pltpu.VMEM((K, 16), jnp.float32),
        pltpu.VMEM_SHARED((ACC_ROWS, 16), jnp.float32),
        pltpu.SemaphoreType.DMA,
        pltpu.SemaphoreType.DMA,
    ],
)
def _sc_degree(dst_hbm, out_hbm, dst_a, dst_b, buf_v, acc_sh, sem_a, sem_b):
    cid = lax.axis_index("c")
    sid = lax.axis_index("s")
    wid = sid * NC + cid
    base = wid * EPW

    def fill(val):
        def step(i, _):
            buf_v[i] = jnp.full((16,), val, jnp.float32)
            return 0
        lax.fori_loop(0, K, step, 0)

    fill(0.0)

    def zero_slice(k, _):
        pltpu.sync_copy(buf_v, acc_sh.at[pl.ds(sid * RPS + k * K, K)])
        return 0

    lax.fori_loop(0, RPS // K, zero_slice, 0)
    fill(1.0)
    plsc.subcore_barrier()

    def chunk(j, _):
        pltpu.sync_copy(dst_hbm.at[pl.ds(base + j * K, K)], dst_a)
        pltpu.sync_copy(buf_v, acc_sh.at[dst_a], add=True)
        return 0

    lax.fori_loop(0, CHUNKS, chunk, 0)
    plsc.subcore_barrier()
    pltpu.sync_copy(
        acc_sh.at[pl.ds(sid * RPS, RPS)],
        out_hbm.at[pl.ds(cid * ACC_ROWS + sid * RPS, RPS)],
    )


@functools.partial(
    pl.kernel,
    out_type=jax.ShapeDtypeStruct((NC * ACC_ROWS, D), jnp.float32),
    mesh=_mesh,
    scratch_types=[
        pltpu.VMEM((K,), jnp.int32),
        pltpu.VMEM((K,), jnp.int32),
        pltpu.VMEM((K,), jnp.int32),
        pltpu.VMEM((K,), jnp.int32),
        pltpu.VMEM((K, D), jnp.float32),
        pltpu.VMEM((K, D), jnp.float32),
        pltpu.VMEM_SHARED((ACC_ROWS, D), jnp.float32),
        pltpu.SemaphoreType.DMA,
        pltpu.SemaphoreType.DMA,
    ],
)
def _sc_scatter(y_hbm, src_hbm, dst_hbm, out_hbm, src_a, src_b, dst_a, dst_b,
                rows_a, rows_b, acc_sh, sem_a, sem_b):
    cid = lax.axis_index("c")
    sid = lax.axis_index("s")
    wid = sid * NC + cid
    base = wid * EPW

    def zero_rows(t, _):
        rows_a[t // 8, pl.ds((t % 8) * 16, 16)] = jnp.zeros((16,), jnp.float32)
        return 0

    lax.fori_loop(0, K * (D // 16), zero_rows, 0)

    def zero_slice(k, _):
        pltpu.sync_copy(rows_a, acc_sh.at[pl.ds(sid * RPS + k * K, K)])
        return 0

    lax.fori_loop(0, RPS // K, zero_slice, 0)

    def load_gather(j, src_v, dst_v, rows_v, sem):
        pltpu.sync_copy(src_hbm.at[pl.ds(base + j * K, K)], src_v)
        pltpu.sync_copy(dst_hbm.at[pl.ds(base + j * K, K)], dst_v)
        pltpu.async_copy(y_hbm.at[src_v], rows_v, sem)

    def consume(src_v, dst_v, rows_v, sem):
        pltpu.make_async_copy(y_hbm.at[src_v], rows_v, sem).wait()
        pltpu.sync_copy(rows_v, acc_sh.at[dst_v], add=True)

    load_gather(0, src_a, dst_a, rows_a, sem_a)
    plsc.subcore_barrier()

    def body(t, _):
        ja = 2 * t
        jb = 2 * t + 1

        @pl.when(jb < CHUNKS)
        def _():
            load_gather(jb, src_b, dst_b, rows_b, sem_b)

        consume(src_a, dst_a, rows_a, sem_a)

        @pl.when(ja + 2 < CHUNKS)
        def _():
            load_gather(ja + 2, src_a, dst_a, rows_a, sem_a)

        @pl.when(jb < CHUNKS)
        def _():
            consume(src_b, dst_b, rows_b, sem_b)

        return 0

    lax.fori_loop(0, (CHUNKS + 1) // 2, body, 0)
    plsc.subcore_barrier()
    pltpu.sync_copy(
        acc_sh.at[pl.ds(sid * RPS, RPS)],
        out_hbm.at[pl.ds(cid * ACC_ROWS + sid * RPS, RPS)],
    )


_ROWS_BLK = 1000
_GRID = N // _ROWS_BLK


def _tc1(x, W1, d0, d1):
    def body(x_ref, w_ref, d0_ref, d1_ref, y_ref):
        dinv = lax.rsqrt(d0_ref[...] + d1_ref[...] + 1.0)
        y_ref[...] = (
            jnp.dot(x_ref[...], w_ref[...], preferred_element_type=jnp.float32)
            * dinv
        )

    return pl.pallas_call(
        body,
        grid=(_GRID,),
        in_specs=[
            pl.BlockSpec((_ROWS_BLK, D), lambda i: (i, 0)),
            pl.BlockSpec((D, D), lambda i: (0, 0)),
            pl.BlockSpec((_ROWS_BLK, 1), lambda i: (i, 0)),
            pl.BlockSpec((_ROWS_BLK, 1), lambda i: (i, 0)),
        ],
        out_specs=pl.BlockSpec((_ROWS_BLK, D), lambda i: (i, 0)),
        out_shape=jax.ShapeDtypeStruct((N, D), jnp.float32),
    )(x, W1, d0, d1)


def _tc2(p0, p1, y1, d0, d1, b1, W2):
    def body(p0_ref, p1_ref, y1_ref, d0_ref, d1_ref, b_ref, w_ref, out_ref):
        dinv = lax.rsqrt(d0_ref[...] + d1_ref[...] + 1.0)
        h = dinv * (p0_ref[...] + p1_ref[...] + y1_ref[...]) + b_ref[...]
        h = jnp.where(h > 0, h, 0.01 * h)
        out_ref[...] = (
            jnp.dot(h, w_ref[...], preferred_element_type=jnp.float32) * dinv
        )

    return pl.pallas_call(
        body,
        grid=(_GRID,),
        in_specs=[
            pl.BlockSpec((_ROWS_BLK, D), lambda i: (i, 0)),
            pl.BlockSpec((_ROWS_BLK, D), lambda i: (i, 0)),
            pl.BlockSpec((_ROWS_BLK, D), lambda i: (i, 0)),
            pl.BlockSpec((_ROWS_BLK, 1), lambda i: (i, 0)),
            pl.BlockSpec((_ROWS_BLK, 1), lambda i: (i, 0)),
            pl.BlockSpec((1, D), lambda i: (0, 0)),
            pl.BlockSpec((D, D), lambda i: (0, 0)),
        ],
        out_specs=pl.BlockSpec((_ROWS_BLK, D), lambda i: (i, 0)),
        out_shape=jax.ShapeDtypeStruct((N, D), jnp.float32),
    )(p0, p1, y1, d0, d1, b1, W2)


def _tc3(q0, q1, y2, d0, d1, b2):
    def body(q0_ref, q1_ref, y2_ref, d0_ref, d1_ref, b_ref, out_ref):
        dinv = lax.rsqrt(d0_ref[...] + d1_ref[...] + 1.0)
        out_ref[...] = dinv * (q0_ref[...] + q1_ref[...] + y2_ref[...]) + b_ref[...]

    return pl.pallas_call(
        body,
        grid=(_GRID,),
        in_specs=[
            pl.BlockSpec((_ROWS_BLK, D), lambda i: (i, 0)),
            pl.BlockSpec((_ROWS_BLK, D), lambda i: (i, 0)),
            pl.BlockSpec((_ROWS_BLK, D), lambda i: (i, 0)),
            pl.BlockSpec((_ROWS_BLK, 1), lambda i: (i, 0)),
            pl.BlockSpec((_ROWS_BLK, 1), lambda i: (i, 0)),
            pl.BlockSpec((1, D), lambda i: (0, 0)),
        ],
        out_specs=pl.BlockSpec((_ROWS_BLK, D), lambda i: (i, 0)),
        out_shape=jax.ShapeDtypeStruct((N, D), jnp.float32),
    )(q0, q1, y2, d0, d1, b2)


def kernel(x, edge_index, W1, b1, W2, b2):
    src = edge_index[0]
    dst = edge_index[1]
    pad = E_PAD - E
    srcp = jnp.concatenate([src, jnp.zeros((pad,), jnp.int32)])
    dstp = jnp.concatenate([dst, jnp.full((pad,), N, jnp.int32)])

    degp = _sc_degree(dstp)
    d0 = degp[:N, :1]
    d1 = degp[ACC_ROWS:ACC_ROWS + N, :1]

    y1 = _tc1(x, W1, d0, d1)

    p = _sc_scatter(y1, srcp, dstp)
    y2 = _tc2(p[:N], p[ACC_ROWS:ACC_ROWS + N], y1, d0, d1,
              b1.reshape(1, D), W2)

    q = _sc_scatter(y2, srcp, dstp)
    return _tc3(q[:N], q[ACC_ROWS:ACC_ROWS + N], y2, d0, d1,
                b2.reshape(1, D))


# speedup vs baseline: 13.3638x; 1.0653x over previous
import functools

import jax
import jax.numpy as jnp
from jax import lax
from jax.experimental import pallas as pl
from jax.experimental.pallas import tpu as pltpu
from jax.experimental.pallas import tpu_sc as plsc

N = 10000
E = 320000
D = 128
NC = 2
NS = 16
NW = NC * NS
K = 128
CHUNKS = 79
EPW = K * CHUNKS
E_PAD = EPW * NW
ACC_ROWS = 10240
RPS = ACC_ROWS // NS

_mesh = plsc.VectorSubcoreMesh(core_axis_name="c", subcore_axis_name="s")


@functools.partial(
    pl.kernel,
    out_type=jax.ShapeDtypeStruct((NC * ACC_ROWS, 16), jnp.float32),
    mesh=_mesh,
    scratch_types=[
        pltpu.VMEM((K,), jnp.int32),
        pltpu.VMEM((K,), jnp.int32),
        pltpu.VMEM((K, 16), jnp.float32),
        pltpu.VMEM_SHARED((ACC_ROWS, 16), jnp.float32),
        pltpu.SemaphoreType.DMA,
        pltpu.SemaphoreType.DMA,
    ],
)
def _sc_degree(dst_hbm, out_hbm, dst_a, dst_b, buf_v, acc_sh, sem_a, sem_b):
    cid = lax.axis_index("c")
    sid = lax.axis_index("s")
    wid = sid * NC + cid
    base = wid * EPW

    def fill(val):
        def step(i, _):
            buf_v[i] = jnp.full((16,), val, jnp.float32)
            return 0
        lax.fori_loop(0, K, step, 0)

    fill(0.0)

    def zero_slice(k, _):
        pltpu.sync_copy(buf_v, acc_sh.at[pl.ds(sid * RPS + k * K, K)])
        return 0

    lax.fori_loop(0, RPS // K, zero_slice, 0)
    fill(1.0)
    plsc.subcore_barrier()

    def chunk(j, _):
        pltpu.sync_copy(dst_hbm.at[pl.ds(base + j * K, K)], dst_a)
        pltpu.sync_copy(buf_v, acc_sh.at[dst_a], add=True)
        return 0

    lax.fori_loop(0, CHUNKS, chunk, 0)
    plsc.subcore_barrier()
    pltpu.sync_copy(
        acc_sh.at[pl.ds(sid * RPS, RPS)],
        out_hbm.at[pl.ds(cid * ACC_ROWS + sid * RPS, RPS)],
    )


@functools.partial(
    pl.kernel,
    out_type=jax.ShapeDtypeStruct((NC * ACC_ROWS, D), jnp.float32),
    mesh=_mesh,
    scratch_types=(
        [pltpu.VMEM((K,), jnp.int32) for _ in range(8)]
        + [pltpu.VMEM((K, D), jnp.float32) for _ in range(2)]
        + [pltpu.VMEM_SHARED((ACC_ROWS, D), jnp.float32)]
        + [pltpu.SemaphoreType.DMA for _ in range(8)]
    ),
)
def _sc_scatter(y_hbm, src_hbm, dst_hbm, out_hbm,
                si0, si1, si2, si3, di0, di1, di2, di3,
                rows_a, rows_b, acc_sh,
                is0, is1, is2, is3, gs0, gs1, ss0, ss1):
    cid = lax.axis_index("c")
    sid = lax.axis_index("s")
    wid = sid * NC + cid
    base = wid * EPW

    src_i = [si0, si1, si2, si3]
    dst_i = [di0, di1, di2, di3]
    isem = [is0, is1, is2, is3]
    rows = [rows_a, rows_b]
    gsem = [gs0, gs1]
    ssem = [ss0, ss1]

    def zero_rows(t, _):
        rows_a[t // 8, pl.ds((t % 8) * 16, 16)] = jnp.zeros((16,), jnp.float32)
        return 0

    lax.fori_loop(0, K * (D // 16), zero_rows, 0)

    def zero_slice(k, _):
        pltpu.sync_copy(rows_a, acc_sh.at[pl.ds(sid * RPS + k * K, K)])
        return 0

    lax.fori_loop(0, RPS // K, zero_slice, 0)

    def iload(j, p):
        pltpu.async_copy(src_hbm.at[pl.ds(base + j * K, K)], src_i[p], isem[p])
        pltpu.async_copy(dst_hbm.at[pl.ds(base + j * K, K)], dst_i[p], isem[p])

    def iwait(j, p):
        pltpu.make_async_copy(src_hbm.at[pl.ds(base + j * K, K)], src_i[p],
                              isem[p]).wait()
        pltpu.make_async_copy(dst_hbm.at[pl.ds(base + j * K, K)], dst_i[p],
                              isem[p]).wait()

    def gstart(p, r):
        pltpu.async_copy(y_hbm.at[src_i[p]], rows[r], gsem[r])

    def gwait(p, r):
        pltpu.make_async_copy(y_hbm.at[src_i[p]], rows[r], gsem[r]).wait()

    def sstart(p, r):
        pltpu.async_copy(rows[r], acc_sh.at[dst_i[p]], ssem[r], add=True)

    def swait(p, r):
        pltpu.make_async_copy(rows[r], acc_sh.at[dst_i[p]], ssem[r]).wait()

    def step(j, o, first=False, do1=True, do2=True):
        p = o % 4
        r = o % 2
        pn = (o + 1) % 4
        rn = (o + 1) % 2
        if do1:
            iwait(j + 1, pn)
        if not first:
            swait((o - 1) % 4, rn)
        if do1:
            gstart(pn, rn)
            if do2:
                iload(j + 2, (o + 2) % 4)
        gwait(p, r)
        sstart(p, r)

    iload(0, 0)
    iload(1, 1)
    iwait(0, 0)
    gstart(0, 0)
    plsc.subcore_barrier()

    QUADS = (CHUNKS - 3) // 4
    step(0, 0, first=True)
    for j in range(1, 4):
        step(j, j)

    def quad(t, _):
        j0 = 4 * t
        for o in range(4):
            step(j0 + o, o)
        return 0

    lax.fori_loop(1, QUADS, quad, 0)
    for j in range(4 * QUADS, CHUNKS):
        step(j, j % 4, do1=(j + 1 < CHUNKS), do2=(j + 2 < CHUNKS))
    swait((CHUNKS - 1) % 4, (CHUNKS - 1) % 2)
    plsc.subcore_barrier()
    pltpu.sync_copy(
        acc_sh.at[pl.ds(sid * RPS, RPS)],
        out_hbm.at[pl.ds(cid * ACC_ROWS + sid * RPS, RPS)],
    )


_ROWS_BLK = 1000
_GRID = N // _ROWS_BLK


def _tc1(x, W1, d0, d1):
    def body(x_ref, w_ref, d0_ref, d1_ref, y_ref):
        dinv = lax.rsqrt(d0_ref[...] + d1_ref[...] + 1.0)
        y_ref[...] = (
            jnp.dot(x_ref[...], w_ref[...], preferred_element_type=jnp.float32)
            * dinv
        )

    return pl.pallas_call(
        body,
        grid=(_GRID,),
        in_specs=[
            pl.BlockSpec((_ROWS_BLK, D), lambda i: (i, 0)),
            pl.BlockSpec((D, D), lambda i: (0, 0)),
            pl.BlockSpec((_ROWS_BLK, 1), lambda i: (i, 0)),
            pl.BlockSpec((_ROWS_BLK, 1), lambda i: (i, 0)),
        ],
        out_specs=pl.BlockSpec((_ROWS_BLK, D), lambda i: (i, 0)),
        out_shape=jax.ShapeDtypeStruct((N, D), jnp.float32),
    )(x, W1, d0, d1)


def _tc2(p0, p1, y1, d0, d1, b1, W2):
    def body(p0_ref, p1_ref, y1_ref, d0_ref, d1_ref, b_ref, w_ref, out_ref):
        dinv = lax.rsqrt(d0_ref[...] + d1_ref[...] + 1.0)
        h = dinv * (p0_ref[...] + p1_ref[...] + y1_ref[...]) + b_ref[...]
        h = jnp.where(h > 0, h, 0.01 * h)
        out_ref[...] = (
            jnp.dot(h, w_ref[...], preferred_element_type=jnp.float32) * dinv
        )

    return pl.pallas_call(
        body,
        grid=(_GRID,),
        in_specs=[
            pl.BlockSpec((_ROWS_BLK, D), lambda i: (i, 0)),
            pl.BlockSpec((_ROWS_BLK, D), lambda i: (i, 0)),
            pl.BlockSpec((_ROWS_BLK, D), lambda i: (i, 0)),
            pl.BlockSpec((_ROWS_BLK, 1), lambda i: (i, 0)),
            pl.BlockSpec((_ROWS_BLK, 1), lambda i: (i, 0)),
            pl.BlockSpec((1, D), lambda i: (0, 0)),
            pl.BlockSpec((D, D), lambda i: (0, 0)),
        ],
        out_specs=pl.BlockSpec((_ROWS_BLK, D), lambda i: (i, 0)),
        out_shape=jax.ShapeDtypeStruct((N, D), jnp.float32),
    )(p0, p1, y1, d0, d1, b1, W2)


def _tc3(q0, q1, y2, d0, d1, b2):
    def body(q0_ref, q1_ref, y2_ref, d0_ref, d1_ref, b_ref, out_ref):
        dinv = lax.rsqrt(d0_ref[...] + d1_ref[...] + 1.0)
        out_ref[...] = dinv * (q0_ref[...] + q1_ref[...] + y2_ref[...]) + b_ref[...]

    return pl.pallas_call(
        body,
        grid=(_GRID,),
        in_specs=[
            pl.BlockSpec((_ROWS_BLK, D), lambda i: (i, 0)),
            pl.BlockSpec((_ROWS_BLK, D), lambda i: (i, 0)),
            pl.BlockSpec((_ROWS_BLK, D), lambda i: (i, 0)),
            pl.BlockSpec((_ROWS_BLK, 1), lambda i: (i, 0)),
            pl.BlockSpec((_ROWS_BLK, 1), lambda i: (i, 0)),
            pl.BlockSpec((1, D), lambda i: (0, 0)),
        ],
        out_specs=pl.BlockSpec((_ROWS_BLK, D), lambda i: (i, 0)),
        out_shape=jax.ShapeDtypeStruct((N, D), jnp.float32),
    )(q0, q1, y2, d0, d1, b2)


def kernel(x, edge_index, W1, b1, W2, b2):
    src = edge_index[0]
    dst = edge_index[1]
    pad = E_PAD - E
    srcp = jnp.concatenate([src, jnp.zeros((pad,), jnp.int32)])
    dstp = jnp.concatenate([dst, jnp.full((pad,), N, jnp.int32)])

    degp = _sc_degree(dstp)
    d0 = degp[:N, :1]
    d1 = degp[ACC_ROWS:ACC_ROWS + N, :1]

    y1 = _tc1(x, W1, d0, d1)

    p = _sc_scatter(y1, srcp, dstp)
    y2 = _tc2(p[:N], p[ACC_ROWS:ACC_ROWS + N], y1, d0, d1,
              b1.reshape(1, D), W2)

    q = _sc_scatter(y2, srcp, dstp)
    return _tc3(q[:N], q[ACC_ROWS:ACC_ROWS + N], y2, d0, d1,
                b2.reshape(1, D))


# speedup vs baseline: 13.8174x; 1.0339x over previous
import functools

import jax
import jax.numpy as jnp
from jax import lax
from jax.experimental import pallas as pl
from jax.experimental.pallas import tpu as pltpu
from jax.experimental.pallas import tpu_sc as plsc

N = 10000
E = 320000
D = 128
NC = 2
NS = 16
NW = NC * NS
K = 128
CHUNKS = 79
EPW = K * CHUNKS
E_PAD = EPW * NW
ACC_ROWS = 10112
RPS = ACC_ROWS // NS

_mesh = plsc.VectorSubcoreMesh(core_axis_name="c", subcore_axis_name="s")


@functools.partial(
    pl.kernel,
    out_type=jax.ShapeDtypeStruct((NC * ACC_ROWS, 16), jnp.float32),
    mesh=_mesh,
    scratch_types=[
        pltpu.VMEM((K,), jnp.int32),
        pltpu.VMEM((K,), jnp.int32),
        pltpu.VMEM((K, 16), jnp.float32),
        pltpu.VMEM_SHARED((ACC_ROWS, 16), jnp.float32),
        pltpu.SemaphoreType.DMA,
        pltpu.SemaphoreType.DMA,
    ],
)
def _sc_degree(dst_hbm, out_hbm, dst_a, dst_b, buf_v, acc_sh, sem_a, sem_b):
    cid = lax.axis_index("c")
    sid = lax.axis_index("s")
    wid = sid * NC + cid
    base = wid * EPW

    def iload(j, dv, sem):
        pltpu.async_copy(dst_hbm.at[pl.ds(base + j * K, K)], dv, sem)

    def iwait(j, dv, sem):
        pltpu.make_async_copy(dst_hbm.at[pl.ds(base + j * K, K)], dv,
                              sem).wait()

    iload(0, dst_a, sem_a)

    def fill(val):
        def step(i, _):
            buf_v[i] = jnp.full((16,), val, jnp.float32)
            return 0
        lax.fori_loop(0, K, step, 0)

    fill(0.0)

    def zero_slice(k, _):
        pltpu.sync_copy(buf_v, acc_sh.at[pl.ds(sid * RPS + k * K, K)])
        return 0

    lax.fori_loop(0, RPS // K, zero_slice, 0)
    pltpu.sync_copy(
        buf_v.at[pl.ds(0, RPS - (RPS // K) * K)],
        acc_sh.at[pl.ds(sid * RPS + (RPS // K) * K, RPS - (RPS // K) * K)],
    )
    fill(1.0)
    plsc.subcore_barrier()

    def body(t, _):
        ja = 2 * t
        jb = 2 * t + 1
        iwait(ja, dst_a, sem_a)
        iload(jb, dst_b, sem_b)
        pltpu.sync_copy(buf_v, acc_sh.at[dst_a], add=True)
        iwait(jb, dst_b, sem_b)
        iload(jb + 1, dst_a, sem_a)
        pltpu.sync_copy(buf_v, acc_sh.at[dst_b], add=True)
        return 0

    lax.fori_loop(0, (CHUNKS - 1) // 2, body, 0)
    iwait(CHUNKS - 1, dst_a, sem_a)
    pltpu.sync_copy(buf_v, acc_sh.at[dst_a], add=True)
    plsc.subcore_barrier()
    pltpu.sync_copy(
        acc_sh.at[pl.ds(sid * RPS, RPS)],
        out_hbm.at[pl.ds(cid * ACC_ROWS + sid * RPS, RPS)],
    )


@functools.partial(
    pl.kernel,
    out_type=jax.ShapeDtypeStruct((NC * ACC_ROWS, D), jnp.float32),
    mesh=_mesh,
    scratch_types=(
        [pltpu.VMEM((K,), jnp.int32) for _ in range(8)]
        + [pltpu.VMEM((K, D), jnp.float32) for _ in range(2)]
        + [pltpu.VMEM_SHARED((ACC_ROWS, D), jnp.float32)]
        + [pltpu.SemaphoreType.DMA for _ in range(8)]
    ),
)
def _sc_scatter(y_hbm, src_hbm, dst_hbm, out_hbm,
                si0, si1, si2, si3, di0, di1, di2, di3,
                rows_a, rows_b, acc_sh,
                is0, is1, is2, is3, gs0, gs1, ss0, ss1):
    cid = lax.axis_index("c")
    sid = lax.axis_index("s")
    wid = sid * NC + cid
    base = wid * EPW

    src_i = [si0, si1, si2, si3]
    dst_i = [di0, di1, di2, di3]
    isem = [is0, is1, is2, is3]
    rows = [rows_a, rows_b]
    gsem = [gs0, gs1]
    ssem = [ss0, ss1]

    def zero_rows(t, _):
        rows_a[t // 8, pl.ds((t % 8) * 16, 16)] = jnp.zeros((16,), jnp.float32)
        return 0

    lax.fori_loop(0, K * (D // 16), zero_rows, 0)

    def zero_slice(k, _):
        pltpu.sync_copy(rows_a, acc_sh.at[pl.ds(sid * RPS + k * K, K)])
        return 0

    lax.fori_loop(0, RPS // K, zero_slice, 0)
    pltpu.sync_copy(
        rows_a.at[pl.ds(0, RPS - (RPS // K) * K)],
        acc_sh.at[pl.ds(sid * RPS + (RPS // K) * K, RPS - (RPS // K) * K)],
    )

    def iload(j, p):
        pltpu.async_copy(src_hbm.at[pl.ds(base + j * K, K)], src_i[p], isem[p])
        pltpu.async_copy(dst_hbm.at[pl.ds(base + j * K, K)], dst_i[p], isem[p])

    def iwait(j, p):
        pltpu.make_async_copy(src_hbm.at[pl.ds(base + j * K, K)], src_i[p],
                              isem[p]).wait()
        pltpu.make_async_copy(dst_hbm.at[pl.ds(base + j * K, K)], dst_i[p],
                              isem[p]).wait()

    def gstart(p, r):
        pltpu.async_copy(y_hbm.at[src_i[p]], rows[r], gsem[r])

    def gwait(p, r):
        pltpu.make_async_copy(y_hbm.at[src_i[p]], rows[r], gsem[r]).wait()

    def sstart(p, r):
        pltpu.async_copy(rows[r], acc_sh.at[dst_i[p]], ssem[r], add=True)

    def swait(p, r):
        pltpu.make_async_copy(rows[r], acc_sh.at[dst_i[p]], ssem[r]).wait()

    def step(j, o, first=False, do1=True, do2=True):
        p = o % 4
        r = o % 2
        pn = (o + 1) % 4
        rn = (o + 1) % 2
        if do1:
            iwait(j + 1, pn)
        if not first:
            swait((o - 1) % 4, rn)
        if do1:
            gstart(pn, rn)
            if do2:
                iload(j + 2, (o + 2) % 4)
        gwait(p, r)
        sstart(p, r)

    iload(0, 0)
    iload(1, 1)
    iwait(0, 0)
    gstart(0, 0)
    plsc.subcore_barrier()

    QUADS = (CHUNKS - 3) // 4
    step(0, 0, first=True)
    for j in range(1, 4):
        step(j, j)

    def quad(t, _):
        j0 = 4 * t
        for o in range(4):
            step(j0 + o, o)
        return 0

    lax.fori_loop(1, QUADS, quad, 0)
    for j in range(4 * QUADS, CHUNKS):
        step(j, j % 4, do1=(j + 1 < CHUNKS), do2=(j + 2 < CHUNKS))
    swait((CHUNKS - 1) % 4, (CHUNKS - 1) % 2)
    plsc.subcore_barrier()
    pltpu.sync_copy(
        acc_sh.at[pl.ds(sid * RPS, RPS)],
        out_hbm.at[pl.ds(cid * ACC_ROWS + sid * RPS, RPS)],
    )


_ROWS_BLK = 1000
_GRID = N // _ROWS_BLK


def _tc1(x, W1, d0, d1):
    def body(x_ref, w_ref, d0_ref, d1_ref, y_ref):
        dinv = lax.rsqrt(d0_ref[...] + d1_ref[...] + 1.0)
        y_ref[...] = (
            jnp.dot(x_ref[...], w_ref[...], preferred_element_type=jnp.float32)
            * dinv
        )

    return pl.pallas_call(
        body,
        grid=(_GRID,),
        in_specs=[
            pl.BlockSpec((_ROWS_BLK, D), lambda i: (i, 0)),
            pl.BlockSpec((D, D), lambda i: (0, 0)),
            pl.BlockSpec((_ROWS_BLK, 1), lambda i: (i, 0)),
            pl.BlockSpec((_ROWS_BLK, 1), lambda i: (i, 0)),
        ],
        out_specs=pl.BlockSpec((_ROWS_BLK, D), lambda i: (i, 0)),
        out_shape=jax.ShapeDtypeStruct((N, D), jnp.float32),
    )(x, W1, d0, d1)


def _tc2(p0, p1, y1, d0, d1, b1, W2):
    def body(p0_ref, p1_ref, y1_ref, d0_ref, d1_ref, b_ref, w_ref, out_ref):
        dinv = lax.rsqrt(d0_ref[...] + d1_ref[...] + 1.0)
        h = dinv * (p0_ref[...] + p1_ref[...] + y1_ref[...]) + b_ref[...]
        h = jnp.where(h > 0, h, 0.01 * h)
        out_ref[...] = (
            jnp.dot(h, w_ref[...], preferred_element_type=jnp.float32) * dinv
        )

    return pl.pallas_call(
        body,
        grid=(_GRID,),
        in_specs=[
            pl.BlockSpec((_ROWS_BLK, D), lambda i: (i, 0)),
            pl.BlockSpec((_ROWS_BLK, D), lambda i: (i, 0)),
            pl.BlockSpec((_ROWS_BLK, D), lambda i: (i, 0)),
            pl.BlockSpec((_ROWS_BLK, 1), lambda i: (i, 0)),
            pl.BlockSpec((_ROWS_BLK, 1), lambda i: (i, 0)),
            pl.BlockSpec((1, D), lambda i: (0, 0)),
            pl.BlockSpec((D, D), lambda i: (0, 0)),
        ],
        out_specs=pl.BlockSpec((_ROWS_BLK, D), lambda i: (i, 0)),
        out_shape=jax.ShapeDtypeStruct((N, D), jnp.float32),
    )(p0, p1, y1, d0, d1, b1, W2)


def _tc3(q0, q1, y2, d0, d1, b2):
    def body(q0_ref, q1_ref, y2_ref, d0_ref, d1_ref, b_ref, out_ref):
        dinv = lax.rsqrt(d0_ref[...] + d1_ref[...] + 1.0)
        out_ref[...] = dinv * (q0_ref[...] + q1_ref[...] + y2_ref[...]) + b_ref[...]

    return pl.pallas_call(
        body,
        grid=(_GRID,),
        in_specs=[
            pl.BlockSpec((_ROWS_BLK, D), lambda i: (i, 0)),
            pl.BlockSpec((_ROWS_BLK, D), lambda i: (i, 0)),
            pl.BlockSpec((_ROWS_BLK, D), lambda i: (i, 0)),
            pl.BlockSpec((_ROWS_BLK, 1), lambda i: (i, 0)),
            pl.BlockSpec((_ROWS_BLK, 1), lambda i: (i, 0)),
            pl.BlockSpec((1, D), lambda i: (0, 0)),
        ],
        out_specs=pl.BlockSpec((_ROWS_BLK, D), lambda i: (i, 0)),
        out_shape=jax.ShapeDtypeStruct((N, D), jnp.float32),
    )(q0, q1, y2, d0, d1, b2)


def kernel(x, edge_index, W1, b1, W2, b2):
    src = edge_index[0]
    dst = edge_index[1]
    pad = E_PAD - E
    srcp = jnp.concatenate([src, jnp.zeros((pad,), jnp.int32)])
    dstp = jnp.concatenate([dst, jnp.full((pad,), N, jnp.int32)])

    degp = _sc_degree(dstp)
    d0 = degp[:N, :1]
    d1 = degp[ACC_ROWS:ACC_ROWS + N, :1]

    y1 = _tc1(x, W1, d0, d1)

    p = _sc_scatter(y1, srcp, dstp)
    y2 = _tc2(p[:N], p[ACC_ROWS:ACC_ROWS + N], y1, d0, d1,
              b1.reshape(1, D), W2)

    q = _sc_scatter(y2, srcp, dstp)
    return _tc3(q[:N], q[ACC_ROWS:ACC_ROWS + N], y2, d0, d1,
                b2.reshape(1, D))
